# Initial kernel scaffold; baseline (speedup 1.0000x reference)
import jax
import jax.numpy as jnp
from jax.experimental import pallas as pl


def kernel(inputs, centers):
    raise NotImplementedError("write your pallas kernel here")


# speedup vs baseline: 18.7862x; 18.7862x over previous
import functools

import jax
import jax.numpy as jnp
from jax import lax
from jax.experimental import pallas as pl
from jax.experimental.pallas import tpu as pltpu
from jax.experimental.pallas import tpu_sc as plsc

NW = 32
BS = 512
NBUF = 2
L = 16


def _sc_body(n, c, rows_per_w, n_iter, x_hbm, params_hbm, out_hbm,
             xall, obuf, offs, cbuf, sem_in, sem_o0, sem_o1):
    cid = lax.axis_index("c")
    sid = lax.axis_index("s")
    wid = sid * 2 + cid
    row0 = wid * rows_per_w
    sems = (sem_o0, sem_o1)

    pltpu.sync_copy(params_hbm, cbuf)
    xcp = pltpu.async_copy(x_hbm.at[pl.ds(row0, rows_per_w)], xall, sem_in)

    lane = lax.iota(jnp.int32, L)
    low = cbuf[pl.ds(0, L)]
    inv_w = cbuf[pl.ds(L, L)]
    lane_c = lane * c
    zf = jnp.zeros((L,), jnp.float32)

    @pl.loop(0, NBUF * BS * c // L)
    def _(k):
        obuf[pl.ds(k * L, L)] = zf

    for b in range(NBUF):
        @pl.loop(0, BS // L)
        def _(g, _b=b):
            offs[pl.ds(_b * BS + g * L, L)] = _b * BS * c + g * (L * c) + lane_c

    xcp.wait()

    def fill(it, b):
        ob_base = b * BS * c

        @pl.loop(0, BS // L)
        def _(g):
            xv = xall[pl.ds(it * BS + g * L, L)]
            t = (xv - low) * inv_w
            ti = t.astype(jnp.int32)
            p = t - ti.astype(jnp.float32)
            ov = offs[pl.ds(b * BS + g * L, L)]
            plsc.store_scatter(obuf, [ov], zf)
            plsc.store_scatter(obuf, [ov + 1], zf)
            o = ob_base + g * (L * c) + lane_c + ti
            plsc.store_scatter(obuf, [o], 1.0 - p)
            plsc.store_scatter(obuf, [o + 1], p)
            offs[pl.ds(b * BS + g * L, L)] = o

        dst = (row0 + it * BS) * c
        pltpu.async_copy(obuf.at[pl.ds(ob_base, BS * c)],
                         out_hbm.at[pl.ds(dst, BS * c)], sems[b])

    def drain(b):
        pltpu.make_async_copy(obuf.at[pl.ds(b * BS * c, BS * c)],
                              out_hbm.at[pl.ds(0, BS * c)], sems[b]).wait()

    for b in range(NBUF):
        fill(b, b)

    @pl.loop(1, n_iter // NBUF)
    def _(j):
        for b in range(NBUF):
            drain(b)
            fill(j * NBUF + b, b)

    for b in range(NBUF):
        drain(b)


def kernel(inputs, centers):
    n = inputs.shape[0]
    c = centers.shape[0]
    rows_per_w = n // NW
    n_iter = rows_per_w // BS

    low = centers[0]
    inv_w = 1.0 / (centers[1] - centers[0])
    params = jnp.concatenate([jnp.broadcast_to(low, (L,)),
                              jnp.broadcast_to(inv_w, (L,))])

    body = functools.partial(_sc_body, n, c, rows_per_w, n_iter)
    f = pl.kernel(
        body,
        out_type=jax.ShapeDtypeStruct((n * c,), jnp.float32),
        mesh=plsc.VectorSubcoreMesh(core_axis_name="c", subcore_axis_name="s"),
        compiler_params=pltpu.CompilerParams(needs_layout_passes=False),
        scratch_types=[
            pltpu.VMEM((rows_per_w,), jnp.float32),
            pltpu.VMEM((NBUF * BS * c,), jnp.float32),
            pltpu.VMEM((NBUF * BS,), jnp.int32),
            pltpu.VMEM((2 * L,), jnp.float32),
            pltpu.SemaphoreType.DMA,
            pltpu.SemaphoreType.DMA,
            pltpu.SemaphoreType.DMA,
        ],
    )
    return f(inputs, params).reshape(n, c)
